# baseline (speedup 1.0000x reference)
import functools

import jax
import jax.numpy as jnp
from jax import lax
from jax.experimental import pallas as pl
from jax.experimental.pallas import tpu as pltpu
from jax.experimental.pallas import tpu_sc as plsc

_SC_PARAMS = pltpu.CompilerParams(needs_layout_passes=False)

NC = 2
NS = 16
LANES = 16
ROW_BLK = 400
EDGE_CHUNK = 125


def _pad_rows(N):
    q = LANES * NS
    return ((N + q - 1) // q) * q


def _deg_kernel(N, E):
    Et = E // (NC * NS)
    NP = _pad_rows(N)
    rpt = NP // NS
    mesh = plsc.VectorSubcoreMesh(core_axis_name="c", subcore_axis_name="s")

    @functools.partial(
        pl.kernel,
        out_type=jax.ShapeDtypeStruct((NC, NP), jnp.float32),
        mesh=mesh,
        scratch_types=[
            pltpu.VMEM((Et,), jnp.int32),
            pltpu.VMEM((Et,), jnp.float32),
            pltpu.VMEM((NP,), jnp.float32),
            pltpu.VMEM((NS, rpt), jnp.float32),
            pltpu.VMEM((rpt,), jnp.float32),
            pltpu.VMEM_SHARED((NS, NP), jnp.float32),
        ],
        compiler_params=_SC_PARAMS,
    )
    def deg_kernel(dst_hbm, ew_hbm, out_hbm, dstv, ewv, degv, red, outb, shared):
        c = lax.axis_index("c")
        s = lax.axis_index("s")
        g0 = (c * NS + s) * Et
        row0 = s * rpt

        pltpu.sync_copy(dst_hbm.at[pl.ds(g0, Et)], dstv)
        pltpu.sync_copy(ew_hbm.at[pl.ds(g0, Et)], ewv)

        zero = jnp.zeros((LANES,), jnp.float32)

        @pl.loop(0, NP // LANES)
        def _(i):
            degv[pl.ds(i * LANES, LANES)] = zero

        @pl.loop(0, Et // LANES)
        def _(g):
            sl = pl.ds(g * LANES, LANES)
            plsc.addupdate_scatter(degv, [dstv[sl]], ewv[sl])

        pltpu.sync_copy(degv, shared.at[s])
        plsc.subcore_barrier()
        for t in range(NS):
            pltpu.sync_copy(shared.at[t, pl.ds(row0, rpt)], red.at[t])

        @pl.loop(0, rpt // LANES)
        def _(j):
            sl = pl.ds(j * LANES, LANES)
            acc = red[0, sl]
            for t in range(1, NS):
                acc = acc + red[t, sl]
            outb[sl] = acc

        pltpu.sync_copy(outb, out_hbm.at[c, pl.ds(row0, rpt)])

    return deg_kernel


def _edge_kernel(N, E, D):
    C = EDGE_CHUNK
    Et = E // (NC * NS)
    n_chunks = Et // C
    NP = _pad_rows(N)
    rpt = NP // NS
    ZR = 8
    mesh = plsc.VectorSubcoreMesh(core_axis_name="c", subcore_axis_name="s")

    @functools.partial(
        pl.kernel,
        out_type=jax.ShapeDtypeStruct((NC, NP, D), jnp.float32),
        mesh=mesh,
        scratch_types=[
            pltpu.VMEM((3, C), jnp.int32),
            pltpu.VMEM((3, C), jnp.int32),
            pltpu.VMEM((C, D), jnp.float32),
            pltpu.VMEM((C, D), jnp.float32),
            pltpu.VMEM((ZR, D), jnp.float32),
            pltpu.VMEM_SHARED((NP, D), jnp.float32),
            pltpu.SemaphoreType.DMA,
            pltpu.SemaphoreType.DMA,
            pltpu.SemaphoreType.DMA,
            pltpu.SemaphoreType.DMA,
        ],
        compiler_params=_SC_PARAMS,
    )
    def edge_kernel(meta_hbm, xs_hbm, out_hbm,
                    meta_a, meta_b, rows_a, rows_b, zbuf, acc,
                    sem_ia, sem_ib, sem_ga, sem_gb):
        c = lax.axis_index("c")
        s = lax.axis_index("s")
        w = c * NS + s
        row0 = s * rpt

        zero = jnp.zeros((LANES,), jnp.float32)
        for r in range(ZR):
            for j in range(D // LANES):
                zbuf[r, pl.ds(j * LANES, LANES)] = zero

        @pl.loop(0, rpt // ZR)
        def _(z):
            pltpu.sync_copy(zbuf, acc.at[pl.ds(row0 + z * ZR, ZR)])

        plsc.subcore_barrier()

        def stage_meta(i, meta, sem):
            pltpu.async_copy(meta_hbm.at[w, i], meta, sem)

        def start_gather(i, meta, rows, sem_i, sem_g):
            pltpu.make_async_copy(meta_hbm.at[w, i], meta, sem_i).wait()
            pltpu.async_copy(xs_hbm.at[meta.at[0]], rows, sem_g)

        stage_meta(0, meta_a, sem_ia)
        stage_meta(1, meta_b, sem_ib)
        start_gather(0, meta_a, rows_a, sem_ia, sem_ga)
        start_gather(1, meta_b, rows_b, sem_ib, sem_gb)

        two = jnp.full((LANES,), 2, jnp.int32)

        def process(i, meta, rows, sem_g):
            pltpu.make_async_copy(xs_hbm.at[meta.at[0]], rows, sem_g).wait()

            @plsc.parallel_loop(0, C, unroll=8)
            def _(e):
                widx = jnp.full((LANES,), e, jnp.int32)
                wv = plsc.bitcast(plsc.load_gather(meta, [two, widx]),
                                  jnp.float32)
                for j in range(D // LANES):
                    sl = pl.ds(j * LANES, LANES)
                    rows[e, sl] = rows[e, sl] * wv

            pltpu.sync_copy(rows, acc.at[meta.at[1]], add=True)

        @pl.loop(0, n_chunks, step=2)
        def _(i):
            process(i, meta_a, rows_a, sem_ga)

            @pl.when(i + 2 < n_chunks)
            def _():
                stage_meta(i + 2, meta_a, sem_ia)

            process(i + 1, meta_b, rows_b, sem_gb)

            @pl.when(i + 2 < n_chunks)
            def _():
                start_gather(i + 2, meta_a, rows_a, sem_ia, sem_ga)

            @pl.when(i + 3 < n_chunks)
            def _():
                stage_meta(i + 3, meta_b, sem_ib)
                start_gather(i + 3, meta_b, rows_b, sem_ib, sem_gb)

        plsc.subcore_barrier()
        pltpu.sync_copy(acc.at[pl.ds(row0, rpt)],
                        out_hbm.at[c, pl.ds(row0, rpt)])

    return edge_kernel


def _mm_body(x_ref, w_ref, o_ref):
    o_ref[...] = jnp.dot(x_ref[...], w_ref[...],
                         preferred_element_type=jnp.float32,
                         precision=lax.Precision.HIGHEST)


def _mm(x, W):
    N, K = x.shape
    M = W.shape[1]
    return pl.pallas_call(
        _mm_body,
        grid=(N // ROW_BLK,),
        in_specs=[
            pl.BlockSpec((ROW_BLK, K), lambda i: (i, 0)),
            pl.BlockSpec((K, M), lambda i: (0, 0)),
        ],
        out_specs=pl.BlockSpec((ROW_BLK, M), lambda i: (i, 0)),
        out_shape=jax.ShapeDtypeStruct((N, M), jnp.float32),
    )(x, W)


def _dis_body(degp_ref, dis_ref):
    deg = degp_ref[0, :] + degp_ref[1, :] + 1.0
    dis_ref[...] = lax.rsqrt(deg)[:, None]


def _dis(degp):
    NP = degp.shape[1]
    return pl.pallas_call(
        _dis_body,
        grid=(1,),
        in_specs=[pl.BlockSpec((NC, NP), lambda i: (0, 0))],
        out_specs=pl.BlockSpec((NP, 1), lambda i: (0, 0)),
        out_shape=jax.ShapeDtypeStruct((NP, 1), jnp.float32),
    )(degp)


def _mm1s_body(x_ref, w_ref, dis_ref, xw_ref, xs_ref):
    xw = jnp.dot(x_ref[...], w_ref[...],
                 preferred_element_type=jnp.float32,
                 precision=lax.Precision.HIGHEST)
    xw_ref[...] = xw
    xs_ref[...] = xw * dis_ref[...]


def _mm1s(x, W, dis):
    N, K = x.shape
    M = W.shape[1]
    return pl.pallas_call(
        _mm1s_body,
        grid=(N // ROW_BLK,),
        in_specs=[
            pl.BlockSpec((ROW_BLK, K), lambda i: (i, 0)),
            pl.BlockSpec((K, M), lambda i: (0, 0)),
            pl.BlockSpec((ROW_BLK, 1), lambda i: (i, 0)),
        ],
        out_specs=[
            pl.BlockSpec((ROW_BLK, M), lambda i: (i, 0)),
            pl.BlockSpec((ROW_BLK, M), lambda i: (i, 0)),
        ],
        out_shape=[
            jax.ShapeDtypeStruct((N, M), jnp.float32),
            jax.ShapeDtypeStruct((N, M), jnp.float32),
        ],
    )(x, W, dis)


def _comb2_body(ep_ref, xw1_ref, dis_ref, b1_ref, w2_ref, xw2_ref, xs2_ref):
    dis = dis_ref[...]
    h = (ep_ref[0] + ep_ref[1]) * dis + xw1_ref[...] * (dis * dis) + b1_ref[...]
    h = jnp.maximum(h, 0.0)
    xw2 = jnp.dot(h, w2_ref[...], preferred_element_type=jnp.float32,
                  precision=lax.Precision.HIGHEST)
    xw2_ref[...] = xw2
    pad = jnp.zeros_like(xw2)
    xs2_ref[...] = jnp.concatenate([xw2 * dis, pad], axis=1)


def _comb2(ep, xw1, dis, b1, W2):
    N, D = xw1.shape
    M = W2.shape[1]
    return pl.pallas_call(
        _comb2_body,
        grid=(N // ROW_BLK,),
        in_specs=[
            pl.BlockSpec((NC, ROW_BLK, ep.shape[2]), lambda i: (0, i, 0)),
            pl.BlockSpec((ROW_BLK, D), lambda i: (i, 0)),
            pl.BlockSpec((ROW_BLK, 1), lambda i: (i, 0)),
            pl.BlockSpec((1, D), lambda i: (0, 0)),
            pl.BlockSpec((D, M), lambda i: (0, 0)),
        ],
        out_specs=[
            pl.BlockSpec((ROW_BLK, M), lambda i: (i, 0)),
            pl.BlockSpec((ROW_BLK, 2 * M), lambda i: (i, 0)),
        ],
        out_shape=[
            jax.ShapeDtypeStruct((N, M), jnp.float32),
            jax.ShapeDtypeStruct((N, 2 * M), jnp.float32),
        ],
    )(ep, xw1, dis, b1, W2)


def _final_body(ep_ref, xw2_ref, dis_ref, b2_ref, o_ref):
    dis = dis_ref[...]
    m_out = xw2_ref.shape[1]
    esum = ep_ref[0, :, :m_out] + ep_ref[1, :, :m_out]
    o = esum * dis + xw2_ref[...] * (dis * dis) + b2_ref[...]
    m = jnp.max(o, axis=1, keepdims=True)
    z = o - m
    o_ref[...] = z - jnp.log(jnp.sum(jnp.exp(z), axis=1, keepdims=True))


def _final(ep, xw2, dis, b2):
    N, M = xw2.shape
    return pl.pallas_call(
        _final_body,
        grid=(N // ROW_BLK,),
        in_specs=[
            pl.BlockSpec((NC, ROW_BLK, ep.shape[2]), lambda i: (0, i, 0)),
            pl.BlockSpec((ROW_BLK, M), lambda i: (i, 0)),
            pl.BlockSpec((ROW_BLK, 1), lambda i: (i, 0)),
            pl.BlockSpec((1, M), lambda i: (0, 0)),
        ],
        out_specs=pl.BlockSpec((ROW_BLK, M), lambda i: (i, 0)),
        out_shape=jax.ShapeDtypeStruct((N, M), jnp.float32),
    )(ep, xw2, dis, b2)


def kernel(x, edge_index, edge_weight, W1, b1, W2, b2):
    N = x.shape[0]
    E = edge_index.shape[1]
    src = edge_index[0].astype(jnp.int32)
    dst = edge_index[1].astype(jnp.int32)
    ew = edge_weight.astype(jnp.float32)

    C = EDGE_CHUNK
    n_chunks = E // (NC * NS) // C
    ew_bits = lax.bitcast_convert_type(ew, jnp.int32)
    meta3 = jnp.stack(
        [src.reshape(NC * NS, n_chunks, C),
         dst.reshape(NC * NS, n_chunks, C),
         ew_bits.reshape(NC * NS, n_chunks, C)], axis=2)

    degp = _deg_kernel(N, E)(dst, ew)
    dis = _dis(degp)
    xw1, xs1 = _mm1s(x, W1, dis)
    ep1 = _edge_kernel(N, E, W1.shape[1])(meta3, xs1)
    xw2, xs2 = _comb2(ep1, xw1, dis, b1.reshape(1, -1), W2)
    ep2 = _edge_kernel(N, E, xs2.shape[1])(meta3, xs2)
    return _final(ep2, xw2, dis, b2.reshape(1, -1))


# speedup vs baseline: 1.2038x; 1.2038x over previous
import functools

import jax
import jax.numpy as jnp
from jax import lax
from jax.experimental import pallas as pl
from jax.experimental.pallas import tpu as pltpu
from jax.experimental.pallas import tpu_sc as plsc

_SC_PARAMS = pltpu.CompilerParams(needs_layout_passes=False)

NC = 2
NS = 16
LANES = 16
ROW_BLK = 400
EDGE_CHUNK = 100


def _pad_rows(N):
    q = LANES * NS
    return ((N + q - 1) // q) * q


def _deg_kernel(N, E):
    Et = E // (NC * NS)
    NP = _pad_rows(N)
    rpt = NP // NS
    mesh = plsc.VectorSubcoreMesh(core_axis_name="c", subcore_axis_name="s")

    @functools.partial(
        pl.kernel,
        out_type=jax.ShapeDtypeStruct((NC, NP), jnp.float32),
        mesh=mesh,
        scratch_types=[
            pltpu.VMEM((Et,), jnp.int32),
            pltpu.VMEM((Et,), jnp.float32),
            pltpu.VMEM((NP,), jnp.float32),
            pltpu.VMEM((NS, rpt), jnp.float32),
            pltpu.VMEM((rpt,), jnp.float32),
            pltpu.VMEM_SHARED((NS, NP), jnp.float32),
        ],
        compiler_params=_SC_PARAMS,
    )
    def deg_kernel(dst_hbm, ew_hbm, out_hbm, dstv, ewv, degv, red, outb, shared):
        c = lax.axis_index("c")
        s = lax.axis_index("s")
        g0 = (c * NS + s) * Et
        row0 = s * rpt

        pltpu.sync_copy(dst_hbm.at[pl.ds(g0, Et)], dstv)
        pltpu.sync_copy(ew_hbm.at[pl.ds(g0, Et)], ewv)

        zero = jnp.zeros((LANES,), jnp.float32)

        @pl.loop(0, NP // LANES)
        def _(i):
            degv[pl.ds(i * LANES, LANES)] = zero

        @pl.loop(0, Et // LANES)
        def _(g):
            sl = pl.ds(g * LANES, LANES)
            plsc.addupdate_scatter(degv, [dstv[sl]], ewv[sl])

        pltpu.sync_copy(degv, shared.at[s])
        plsc.subcore_barrier()
        for t in range(NS):
            pltpu.sync_copy(shared.at[t, pl.ds(row0, rpt)], red.at[t])

        @pl.loop(0, rpt // LANES)
        def _(j):
            sl = pl.ds(j * LANES, LANES)
            acc = red[0, sl]
            for t in range(1, NS):
                acc = acc + red[t, sl]
            outb[sl] = acc

        pltpu.sync_copy(outb, out_hbm.at[c, pl.ds(row0, rpt)])

    return deg_kernel


def _edge_kernel(N, E, D):
    C = EDGE_CHUNK
    Et = E // (NC * NS)
    n_chunks = Et // C
    NP = _pad_rows(N)
    rpt = NP // NS
    ZR = 8
    mesh = plsc.VectorSubcoreMesh(core_axis_name="c", subcore_axis_name="s")

    NB = 3

    @functools.partial(
        pl.kernel,
        out_type=jax.ShapeDtypeStruct((NC, NP, D), jnp.float32),
        mesh=mesh,
        scratch_types=(
            [pltpu.VMEM((3, C), jnp.int32) for _ in range(NB)]
            + [pltpu.VMEM((C, D), jnp.float32) for _ in range(NB)]
            + [pltpu.VMEM((ZR, D), jnp.float32),
               pltpu.VMEM_SHARED((NP, D), jnp.float32)]
            + [pltpu.SemaphoreType.DMA for _ in range(3 * NB)]
        ),
        compiler_params=_SC_PARAMS,
    )
    def edge_kernel(meta_hbm, xs_hbm, out_hbm, *refs):
        metas = refs[:NB]
        rows = refs[NB:2 * NB]
        zbuf = refs[2 * NB]
        acc = refs[2 * NB + 1]
        sem_m = refs[2 * NB + 2:2 * NB + 2 + NB]
        sem_g = refs[2 * NB + 2 + NB:2 * NB + 2 + 2 * NB]
        sem_s = refs[2 * NB + 2 + 2 * NB:]

        c = lax.axis_index("c")
        s = lax.axis_index("s")
        w = c * NS + s
        row0 = s * rpt

        zero = jnp.zeros((LANES,), jnp.float32)
        for r in range(ZR):
            for j in range(D // LANES):
                zbuf[r, pl.ds(j * LANES, LANES)] = zero

        @pl.loop(0, rpt // ZR)
        def _(z):
            pltpu.sync_copy(zbuf, acc.at[pl.ds(row0 + z * ZR, ZR)])

        plsc.subcore_barrier()

        def stage(j, k):
            pltpu.async_copy(meta_hbm.at[w, j], metas[k], sem_m[k])

        def gather(j, k):
            pltpu.make_async_copy(meta_hbm.at[w, j], metas[k], sem_m[k]).wait()
            pltpu.async_copy(xs_hbm.at[metas[k].at[0]], rows[k], sem_g[k])

        two = jnp.full((LANES,), 2, jnp.int32)

        def process(j, k):
            pltpu.make_async_copy(xs_hbm.at[metas[k].at[0]], rows[k],
                                  sem_g[k]).wait()

            @plsc.parallel_loop(0, C, unroll=4)
            def _(e):
                widx = jnp.full((LANES,), e, jnp.int32)
                wv = plsc.bitcast(plsc.load_gather(metas[k], [two, widx]),
                                  jnp.float32)
                for jj in range(D // LANES):
                    sl = pl.ds(jj * LANES, LANES)
                    rows[k][e, sl] = rows[k][e, sl] * wv

            pltpu.async_copy(rows[k], acc.at[metas[k].at[1]], sem_s[k],
                             add=True)

        def wait_scatter(k):
            pltpu.make_async_copy(rows[k], acc.at[metas[k].at[1]],
                                  sem_s[k]).wait()

        def refill(i, k):
            jr = i + k + NB

            @pl.when(jr < n_chunks)
            def _():
                wait_scatter(k)
                stage(jr, k)
                gather(jr, k)

        for k in range(NB):
            stage(k, k)
        for k in range(NB):
            gather(k, k)

        @pl.loop(0, n_chunks, step=NB)
        def _(i):
            @pl.when(i < n_chunks)
            def _():
                process(i, 0)

            @pl.when(i + 1 < n_chunks)
            def _():
                process(i + 1, 1)

            refill(i, 0)

            @pl.when(i + 2 < n_chunks)
            def _():
                process(i + 2, 2)

            refill(i, 1)
            refill(i, 2)

        for k in range(NB):
            wait_scatter(k)

        plsc.subcore_barrier()
        pltpu.sync_copy(acc.at[pl.ds(row0, rpt)],
                        out_hbm.at[c, pl.ds(row0, rpt)])

    return edge_kernel


def _mm_body(x_ref, w_ref, o_ref):
    o_ref[...] = jnp.dot(x_ref[...], w_ref[...],
                         preferred_element_type=jnp.float32,
                         precision=lax.Precision.HIGHEST)


def _mm(x, W):
    N, K = x.shape
    M = W.shape[1]
    return pl.pallas_call(
        _mm_body,
        grid=(N // ROW_BLK,),
        in_specs=[
            pl.BlockSpec((ROW_BLK, K), lambda i: (i, 0)),
            pl.BlockSpec((K, M), lambda i: (0, 0)),
        ],
        out_specs=pl.BlockSpec((ROW_BLK, M), lambda i: (i, 0)),
        out_shape=jax.ShapeDtypeStruct((N, M), jnp.float32),
    )(x, W)


def _dis_body(degp_ref, dis_ref):
    deg = degp_ref[0, :] + degp_ref[1, :] + 1.0
    dis_ref[...] = lax.rsqrt(deg)[:, None]


def _dis(degp):
    NP = degp.shape[1]
    return pl.pallas_call(
        _dis_body,
        grid=(1,),
        in_specs=[pl.BlockSpec((NC, NP), lambda i: (0, 0))],
        out_specs=pl.BlockSpec((NP, 1), lambda i: (0, 0)),
        out_shape=jax.ShapeDtypeStruct((NP, 1), jnp.float32),
    )(degp)


def _mm1s_body(x_ref, w_ref, dis_ref, xw_ref, xs_ref):
    xw = jnp.dot(x_ref[...], w_ref[...],
                 preferred_element_type=jnp.float32,
                 precision=lax.Precision.HIGHEST)
    xw_ref[...] = xw
    xs_ref[...] = xw * dis_ref[...]


def _mm1s(x, W, dis):
    N, K = x.shape
    M = W.shape[1]
    return pl.pallas_call(
        _mm1s_body,
        grid=(N // ROW_BLK,),
        in_specs=[
            pl.BlockSpec((ROW_BLK, K), lambda i: (i, 0)),
            pl.BlockSpec((K, M), lambda i: (0, 0)),
            pl.BlockSpec((ROW_BLK, 1), lambda i: (i, 0)),
        ],
        out_specs=[
            pl.BlockSpec((ROW_BLK, M), lambda i: (i, 0)),
            pl.BlockSpec((ROW_BLK, M), lambda i: (i, 0)),
        ],
        out_shape=[
            jax.ShapeDtypeStruct((N, M), jnp.float32),
            jax.ShapeDtypeStruct((N, M), jnp.float32),
        ],
    )(x, W, dis)


def _comb2_body(ep_ref, xw1_ref, dis_ref, b1_ref, w2_ref, xw2_ref, xs2_ref):
    dis = dis_ref[...]
    h = (ep_ref[0] + ep_ref[1]) * dis + xw1_ref[...] * (dis * dis) + b1_ref[...]
    h = jnp.maximum(h, 0.0)
    xw2 = jnp.dot(h, w2_ref[...], preferred_element_type=jnp.float32,
                  precision=lax.Precision.HIGHEST)
    xw2_ref[...] = xw2
    pad = jnp.zeros_like(xw2)
    xs2_ref[...] = jnp.concatenate([xw2 * dis, pad], axis=1)


def _comb2(ep, xw1, dis, b1, W2):
    N, D = xw1.shape
    M = W2.shape[1]
    return pl.pallas_call(
        _comb2_body,
        grid=(N // ROW_BLK,),
        in_specs=[
            pl.BlockSpec((NC, ROW_BLK, ep.shape[2]), lambda i: (0, i, 0)),
            pl.BlockSpec((ROW_BLK, D), lambda i: (i, 0)),
            pl.BlockSpec((ROW_BLK, 1), lambda i: (i, 0)),
            pl.BlockSpec((1, D), lambda i: (0, 0)),
            pl.BlockSpec((D, M), lambda i: (0, 0)),
        ],
        out_specs=[
            pl.BlockSpec((ROW_BLK, M), lambda i: (i, 0)),
            pl.BlockSpec((ROW_BLK, 2 * M), lambda i: (i, 0)),
        ],
        out_shape=[
            jax.ShapeDtypeStruct((N, M), jnp.float32),
            jax.ShapeDtypeStruct((N, 2 * M), jnp.float32),
        ],
    )(ep, xw1, dis, b1, W2)


def _final_body(ep_ref, xw2_ref, dis_ref, b2_ref, o_ref):
    dis = dis_ref[...]
    m_out = xw2_ref.shape[1]
    esum = ep_ref[0, :, :m_out] + ep_ref[1, :, :m_out]
    o = esum * dis + xw2_ref[...] * (dis * dis) + b2_ref[...]
    m = jnp.max(o, axis=1, keepdims=True)
    z = o - m
    o_ref[...] = z - jnp.log(jnp.sum(jnp.exp(z), axis=1, keepdims=True))


def _final(ep, xw2, dis, b2):
    N, M = xw2.shape
    return pl.pallas_call(
        _final_body,
        grid=(N // ROW_BLK,),
        in_specs=[
            pl.BlockSpec((NC, ROW_BLK, ep.shape[2]), lambda i: (0, i, 0)),
            pl.BlockSpec((ROW_BLK, M), lambda i: (i, 0)),
            pl.BlockSpec((ROW_BLK, 1), lambda i: (i, 0)),
            pl.BlockSpec((1, M), lambda i: (0, 0)),
        ],
        out_specs=pl.BlockSpec((ROW_BLK, M), lambda i: (i, 0)),
        out_shape=jax.ShapeDtypeStruct((N, M), jnp.float32),
    )(ep, xw2, dis, b2)


def kernel(x, edge_index, edge_weight, W1, b1, W2, b2):
    N = x.shape[0]
    E = edge_index.shape[1]
    src = edge_index[0].astype(jnp.int32)
    dst = edge_index[1].astype(jnp.int32)
    ew = edge_weight.astype(jnp.float32)

    C = EDGE_CHUNK
    n_chunks = E // (NC * NS) // C
    ew_bits = lax.bitcast_convert_type(ew, jnp.int32)
    meta3 = jnp.stack(
        [src.reshape(NC * NS, n_chunks, C),
         dst.reshape(NC * NS, n_chunks, C),
         ew_bits.reshape(NC * NS, n_chunks, C)], axis=2)

    degp = _deg_kernel(N, E)(dst, ew)
    dis = _dis(degp)
    xw1, xs1 = _mm1s(x, W1, dis)
    ep1 = _edge_kernel(N, E, W1.shape[1])(meta3, xs1)
    xw2, xs2 = _comb2(ep1, xw1, dis, b1.reshape(1, -1), W2)
    ep2 = _edge_kernel(N, E, xs2.shape[1])(meta3, xs2)
    return _final(ep2, xw2, dis, b2.reshape(1, -1))


# speedup vs baseline: 1.3456x; 1.1178x over previous
import functools

import jax
import jax.numpy as jnp
from jax import lax
from jax.experimental import pallas as pl
from jax.experimental.pallas import tpu as pltpu
from jax.experimental.pallas import tpu_sc as plsc

_SC_PARAMS = pltpu.CompilerParams(needs_layout_passes=False)

NC = 2
NS = 16
LANES = 16
ROW_BLK = 400
EDGE_CHUNK = 100


def _pad_rows(N):
    q = LANES * NS
    return ((N + q - 1) // q) * q


def _deg_kernel(N, E):
    Et = E // (NC * NS)
    NP = _pad_rows(N)
    rpt = NP // NS
    mesh = plsc.VectorSubcoreMesh(core_axis_name="c", subcore_axis_name="s")

    @functools.partial(
        pl.kernel,
        out_type=jax.ShapeDtypeStruct((NC, NP), jnp.float32),
        mesh=mesh,
        scratch_types=[
            pltpu.VMEM((Et,), jnp.int32),
            pltpu.VMEM((Et,), jnp.float32),
            pltpu.VMEM((NP,), jnp.float32),
            pltpu.VMEM((NS, rpt), jnp.float32),
            pltpu.VMEM((rpt,), jnp.float32),
            pltpu.VMEM_SHARED((NS, NP), jnp.float32),
        ],
        compiler_params=_SC_PARAMS,
    )
    def deg_kernel(dst_hbm, ew_hbm, out_hbm, dstv, ewv, degv, red, outb, shared):
        c = lax.axis_index("c")
        s = lax.axis_index("s")
        g0 = (c * NS + s) * Et
        row0 = s * rpt

        pltpu.sync_copy(dst_hbm.at[pl.ds(g0, Et)], dstv)
        pltpu.sync_copy(ew_hbm.at[pl.ds(g0, Et)], ewv)

        zero = jnp.zeros((LANES,), jnp.float32)

        @pl.loop(0, NP // LANES)
        def _(i):
            degv[pl.ds(i * LANES, LANES)] = zero

        @pl.loop(0, Et // LANES)
        def _(g):
            sl = pl.ds(g * LANES, LANES)
            plsc.addupdate_scatter(degv, [dstv[sl]], ewv[sl])

        pltpu.sync_copy(degv, shared.at[s])
        plsc.subcore_barrier()
        for t in range(NS):
            pltpu.sync_copy(shared.at[t, pl.ds(row0, rpt)], red.at[t])

        @pl.loop(0, rpt // LANES)
        def _(j):
            sl = pl.ds(j * LANES, LANES)
            acc = red[0, sl]
            for t in range(1, NS):
                acc = acc + red[t, sl]
            outb[sl] = acc

        pltpu.sync_copy(outb, out_hbm.at[c, pl.ds(row0, rpt)])

    return deg_kernel


def _edge_kernel(N, E, D):
    C = EDGE_CHUNK
    Et = E // (NC * NS)
    n_chunks = Et // C
    NP = _pad_rows(N)
    rpt = NP // NS
    ZR = 8
    mesh = plsc.VectorSubcoreMesh(core_axis_name="c", subcore_axis_name="s")

    NB = 3
    NM = 2 * NB

    @functools.partial(
        pl.kernel,
        out_type=jax.ShapeDtypeStruct((NC, NP, D), jnp.float32),
        mesh=mesh,
        scratch_types=(
            [pltpu.VMEM((3, C), jnp.int32) for _ in range(NM)]
            + [pltpu.VMEM((C, D), jnp.float32) for _ in range(NB)]
            + [pltpu.VMEM((ZR, D), jnp.float32),
               pltpu.VMEM_SHARED((NP, D), jnp.float32)]
            + [pltpu.SemaphoreType.DMA for _ in range(NM + 2 * NB)]
        ),
        compiler_params=_SC_PARAMS,
    )
    def edge_kernel(meta_hbm, xs_hbm, out_hbm, *refs):
        metas = refs[:NM]
        rows = refs[NM:NM + NB]
        zbuf = refs[NM + NB]
        acc = refs[NM + NB + 1]
        sem_m = refs[NM + NB + 2:NM + NB + 2 + NM]
        sem_g = refs[NM + NB + 2 + NM:NM + NB + 2 + NM + NB]
        sem_s = refs[NM + NB + 2 + NM + NB:]

        c = lax.axis_index("c")
        s = lax.axis_index("s")
        w = c * NS + s
        row0 = s * rpt

        zero = jnp.zeros((LANES,), jnp.float32)
        for r in range(ZR):
            for j in range(D // LANES):
                zbuf[r, pl.ds(j * LANES, LANES)] = zero

        @pl.loop(0, rpt // ZR)
        def _(z):
            pltpu.sync_copy(zbuf, acc.at[pl.ds(row0 + z * ZR, ZR)])

        plsc.subcore_barrier()

        def stage(j, m):
            pltpu.async_copy(meta_hbm.at[w, j], metas[m], sem_m[m])

        def gather(j, m, r):
            pltpu.make_async_copy(meta_hbm.at[w, j], metas[m], sem_m[m]).wait()
            pltpu.async_copy(xs_hbm.at[metas[m].at[0]], rows[r], sem_g[r])

        two = jnp.full((LANES,), 2, jnp.int32)

        def process(j, m, r):
            pltpu.make_async_copy(xs_hbm.at[metas[m].at[0]], rows[r],
                                  sem_g[r]).wait()

            @plsc.parallel_loop(0, C, unroll=4)
            def _(e):
                widx = jnp.full((LANES,), e, jnp.int32)
                wv = plsc.bitcast(plsc.load_gather(metas[m], [two, widx]),
                                  jnp.float32)
                for jj in range(D // LANES):
                    sl = pl.ds(jj * LANES, LANES)
                    rows[r][e, sl] = rows[r][e, sl] * wv

            pltpu.async_copy(rows[r], acc.at[metas[m].at[1]], sem_s[r],
                             add=True)

        def wait_scatter(r):
            pltpu.make_async_copy(rows[r], acc.at[metas[0].at[1]],
                                  sem_s[r]).wait()

        for m in range(NM):
            stage(m, m)
        for r in range(NB):
            gather(r, r, r)

        @pl.loop(0, n_chunks, step=NM)
        def _(i):
            for t in range(NM):
                j = i + t

                @pl.when(j < n_chunks)
                def _(j=j, t=t):
                    process(j, t, t % NB)

                if t >= 1:
                    jd = i + t - 1
                    jr = jd + NB
                    js = jr + NB

                    @pl.when(jd < n_chunks)
                    def _(jd=jd, t=t):
                        wait_scatter((t - 1) % NB)

                    @pl.when(jr < n_chunks)
                    def _(jr=jr, t=t):
                        gather(jr, (t - 1 + NB) % NM, (t - 1) % NB)

                    @pl.when(js < n_chunks)
                    def _(js=js, t=t):
                        stage(js, (t - 1) % NM)

            jd = i + NM - 1
            jr = jd + NB
            js = jr + NB

            @pl.when(jd < n_chunks)
            def _():
                wait_scatter((NM - 1) % NB)

            @pl.when(jr < n_chunks)
            def _():
                gather(jr, (NM - 1 + NB) % NM, (NM - 1) % NB)

            @pl.when(js < n_chunks)
            def _():
                stage(js, (NM - 1) % NM)

        plsc.subcore_barrier()
        pltpu.sync_copy(acc.at[pl.ds(row0, rpt)],
                        out_hbm.at[c, pl.ds(row0, rpt)])

    return edge_kernel


def _mm_body(x_ref, w_ref, o_ref):
    o_ref[...] = jnp.dot(x_ref[...], w_ref[...],
                         preferred_element_type=jnp.float32,
                         precision=lax.Precision.HIGHEST)


def _mm(x, W):
    N, K = x.shape
    M = W.shape[1]
    return pl.pallas_call(
        _mm_body,
        grid=(N // ROW_BLK,),
        in_specs=[
            pl.BlockSpec((ROW_BLK, K), lambda i: (i, 0)),
            pl.BlockSpec((K, M), lambda i: (0, 0)),
        ],
        out_specs=pl.BlockSpec((ROW_BLK, M), lambda i: (i, 0)),
        out_shape=jax.ShapeDtypeStruct((N, M), jnp.float32),
    )(x, W)


def _dis_body(degp_ref, dis_ref):
    deg = degp_ref[0, :] + degp_ref[1, :] + 1.0
    dis_ref[...] = lax.rsqrt(deg)[:, None]


def _dis(degp):
    NP = degp.shape[1]
    return pl.pallas_call(
        _dis_body,
        grid=(1,),
        in_specs=[pl.BlockSpec((NC, NP), lambda i: (0, 0))],
        out_specs=pl.BlockSpec((NP, 1), lambda i: (0, 0)),
        out_shape=jax.ShapeDtypeStruct((NP, 1), jnp.float32),
    )(degp)


def _mm1s_body(x_ref, w_ref, dis_ref, xw_ref, xs_ref):
    xw = jnp.dot(x_ref[...], w_ref[...],
                 preferred_element_type=jnp.float32,
                 precision=lax.Precision.HIGHEST)
    xw_ref[...] = xw
    xs_ref[...] = xw * dis_ref[...]


def _mm1s(x, W, dis):
    N, K = x.shape
    M = W.shape[1]
    return pl.pallas_call(
        _mm1s_body,
        grid=(N // ROW_BLK,),
        in_specs=[
            pl.BlockSpec((ROW_BLK, K), lambda i: (i, 0)),
            pl.BlockSpec((K, M), lambda i: (0, 0)),
            pl.BlockSpec((ROW_BLK, 1), lambda i: (i, 0)),
        ],
        out_specs=[
            pl.BlockSpec((ROW_BLK, M), lambda i: (i, 0)),
            pl.BlockSpec((ROW_BLK, M), lambda i: (i, 0)),
        ],
        out_shape=[
            jax.ShapeDtypeStruct((N, M), jnp.float32),
            jax.ShapeDtypeStruct((N, M), jnp.float32),
        ],
    )(x, W, dis)


def _comb2_body(ep_ref, xw1_ref, dis_ref, b1_ref, w2_ref, xw2_ref, xs2_ref):
    dis = dis_ref[...]
    h = (ep_ref[0] + ep_ref[1]) * dis + xw1_ref[...] * (dis * dis) + b1_ref[...]
    h = jnp.maximum(h, 0.0)
    xw2 = jnp.dot(h, w2_ref[...], preferred_element_type=jnp.float32,
                  precision=lax.Precision.HIGHEST)
    xw2_ref[...] = xw2
    pad = jnp.zeros_like(xw2)
    xs2_ref[...] = jnp.concatenate([xw2 * dis, pad], axis=1)


def _comb2(ep, xw1, dis, b1, W2):
    N, D = xw1.shape
    M = W2.shape[1]
    return pl.pallas_call(
        _comb2_body,
        grid=(N // ROW_BLK,),
        in_specs=[
            pl.BlockSpec((NC, ROW_BLK, ep.shape[2]), lambda i: (0, i, 0)),
            pl.BlockSpec((ROW_BLK, D), lambda i: (i, 0)),
            pl.BlockSpec((ROW_BLK, 1), lambda i: (i, 0)),
            pl.BlockSpec((1, D), lambda i: (0, 0)),
            pl.BlockSpec((D, M), lambda i: (0, 0)),
        ],
        out_specs=[
            pl.BlockSpec((ROW_BLK, M), lambda i: (i, 0)),
            pl.BlockSpec((ROW_BLK, 2 * M), lambda i: (i, 0)),
        ],
        out_shape=[
            jax.ShapeDtypeStruct((N, M), jnp.float32),
            jax.ShapeDtypeStruct((N, 2 * M), jnp.float32),
        ],
    )(ep, xw1, dis, b1, W2)


def _final_body(ep_ref, xw2_ref, dis_ref, b2_ref, o_ref):
    dis = dis_ref[...]
    m_out = xw2_ref.shape[1]
    esum = ep_ref[0, :, :m_out] + ep_ref[1, :, :m_out]
    o = esum * dis + xw2_ref[...] * (dis * dis) + b2_ref[...]
    m = jnp.max(o, axis=1, keepdims=True)
    z = o - m
    o_ref[...] = z - jnp.log(jnp.sum(jnp.exp(z), axis=1, keepdims=True))


def _final(ep, xw2, dis, b2):
    N, M = xw2.shape
    return pl.pallas_call(
        _final_body,
        grid=(N // ROW_BLK,),
        in_specs=[
            pl.BlockSpec((NC, ROW_BLK, ep.shape[2]), lambda i: (0, i, 0)),
            pl.BlockSpec((ROW_BLK, M), lambda i: (i, 0)),
            pl.BlockSpec((ROW_BLK, 1), lambda i: (i, 0)),
            pl.BlockSpec((1, M), lambda i: (0, 0)),
        ],
        out_specs=pl.BlockSpec((ROW_BLK, M), lambda i: (i, 0)),
        out_shape=jax.ShapeDtypeStruct((N, M), jnp.float32),
    )(ep, xw2, dis, b2)


def kernel(x, edge_index, edge_weight, W1, b1, W2, b2):
    N = x.shape[0]
    E = edge_index.shape[1]
    src = edge_index[0].astype(jnp.int32)
    dst = edge_index[1].astype(jnp.int32)
    ew = edge_weight.astype(jnp.float32)

    C = EDGE_CHUNK
    n_chunks = E // (NC * NS) // C
    ew_bits = lax.bitcast_convert_type(ew, jnp.int32)
    meta3 = jnp.stack(
        [src.reshape(NC * NS, n_chunks, C),
         dst.reshape(NC * NS, n_chunks, C),
         ew_bits.reshape(NC * NS, n_chunks, C)], axis=2)

    degp = _deg_kernel(N, E)(dst, ew)
    dis = _dis(degp)
    xw1, xs1 = _mm1s(x, W1, dis)
    ep1 = _edge_kernel(N, E, W1.shape[1])(meta3, xs1)
    xw2, xs2 = _comb2(ep1, xw1, dis, b1.reshape(1, -1), W2)
    ep2 = _edge_kernel(N, E, xs2.shape[1])(meta3, xs2)
    return _final(ep2, xw2, dis, b2.reshape(1, -1))


# speedup vs baseline: 1.3628x; 1.0127x over previous
import functools

import jax
import jax.numpy as jnp
from jax import lax
from jax.experimental import pallas as pl
from jax.experimental.pallas import tpu as pltpu
from jax.experimental.pallas import tpu_sc as plsc

_SC_PARAMS = pltpu.CompilerParams(needs_layout_passes=False)

NC = 2
NS = 16
LANES = 16
ROW_BLK = 400
EDGE_CHUNK = 100


def _pad_rows(N):
    q = LANES * NS
    return ((N + q - 1) // q) * q


def _deg_kernel(N, E):
    Et = E // (NC * NS)
    NP = _pad_rows(N)
    rpt = NP // NS
    mesh = plsc.VectorSubcoreMesh(core_axis_name="c", subcore_axis_name="s")

    @functools.partial(
        pl.kernel,
        out_type=jax.ShapeDtypeStruct((NC, NP), jnp.float32),
        mesh=mesh,
        scratch_types=[
            pltpu.VMEM((Et,), jnp.int32),
            pltpu.VMEM((Et,), jnp.float32),
            pltpu.VMEM((NP,), jnp.float32),
            pltpu.VMEM((NS, rpt), jnp.float32),
            pltpu.VMEM((rpt,), jnp.float32),
            pltpu.VMEM_SHARED((NS, NP), jnp.float32),
        ],
        compiler_params=_SC_PARAMS,
    )
    def deg_kernel(dst_hbm, ew_hbm, out_hbm, dstv, ewv, degv, red, outb, shared):
        c = lax.axis_index("c")
        s = lax.axis_index("s")
        g0 = (c * NS + s) * Et
        row0 = s * rpt

        pltpu.sync_copy(dst_hbm.at[pl.ds(g0, Et)], dstv)
        pltpu.sync_copy(ew_hbm.at[pl.ds(g0, Et)], ewv)

        zero = jnp.zeros((LANES,), jnp.float32)

        @pl.loop(0, NP // LANES)
        def _(i):
            degv[pl.ds(i * LANES, LANES)] = zero

        @pl.loop(0, Et // LANES)
        def _(g):
            sl = pl.ds(g * LANES, LANES)
            plsc.addupdate_scatter(degv, [dstv[sl]], ewv[sl])

        pltpu.sync_copy(degv, shared.at[s])
        plsc.subcore_barrier()
        for t in range(NS):
            pltpu.sync_copy(shared.at[t, pl.ds(row0, rpt)], red.at[t])

        @pl.loop(0, rpt // LANES)
        def _(j):
            sl = pl.ds(j * LANES, LANES)
            acc = red[0, sl]
            for t in range(1, NS):
                acc = acc + red[t, sl]
            outb[sl] = acc

        pltpu.sync_copy(outb, out_hbm.at[c, pl.ds(row0, rpt)])

    return deg_kernel


def _edge_kernel(N, E, D):
    C = EDGE_CHUNK
    Et = E // (NC * NS)
    n_chunks = Et // C
    NP = _pad_rows(N)
    rpt = NP // NS
    ZR = 8
    mesh = plsc.VectorSubcoreMesh(core_axis_name="c", subcore_axis_name="s")

    NB = 3
    NM = 2 * NB

    @functools.partial(
        pl.kernel,
        out_type=jax.ShapeDtypeStruct((NC, NP, D), jnp.float32),
        mesh=mesh,
        scratch_types=(
            [pltpu.VMEM((3, C), jnp.int32) for _ in range(NM)]
            + [pltpu.VMEM((C, D), jnp.float32) for _ in range(NB)]
            + [pltpu.VMEM((ZR, D), jnp.float32),
               pltpu.VMEM_SHARED((NP, D), jnp.float32)]
            + [pltpu.SemaphoreType.DMA for _ in range(NM + 2 * NB)]
        ),
        compiler_params=_SC_PARAMS,
    )
    def edge_kernel(meta_hbm, xs_hbm, out_hbm, *refs):
        metas = refs[:NM]
        rows = refs[NM:NM + NB]
        zbuf = refs[NM + NB]
        acc = refs[NM + NB + 1]
        sem_m = refs[NM + NB + 2:NM + NB + 2 + NM]
        sem_g = refs[NM + NB + 2 + NM:NM + NB + 2 + NM + NB]
        sem_s = refs[NM + NB + 2 + NM + NB:]

        c = lax.axis_index("c")
        s = lax.axis_index("s")
        w = c * NS + s
        row0 = s * rpt

        zero = jnp.zeros((LANES,), jnp.float32)
        for r in range(ZR):
            for j in range(D // LANES):
                zbuf[r, pl.ds(j * LANES, LANES)] = zero

        @pl.loop(0, rpt // ZR)
        def _(z):
            pltpu.sync_copy(zbuf, acc.at[pl.ds(row0 + z * ZR, ZR)])

        plsc.subcore_barrier()

        def stage(j, m):
            pltpu.async_copy(meta_hbm.at[w, j], metas[m], sem_m[m])

        def gather(j, m, r):
            pltpu.make_async_copy(meta_hbm.at[w, j], metas[m], sem_m[m]).wait()
            pltpu.async_copy(xs_hbm.at[metas[m].at[0]], rows[r], sem_g[r])

        two = jnp.full((LANES,), 2, jnp.int32)

        def process(j, m, r):
            pltpu.make_async_copy(xs_hbm.at[metas[m].at[0]], rows[r],
                                  sem_g[r]).wait()

            @plsc.parallel_loop(0, C, unroll=4)
            def _(e):
                widx = jnp.full((LANES,), e, jnp.int32)
                wv = plsc.bitcast(plsc.load_gather(metas[m], [two, widx]),
                                  jnp.float32)
                for jj in range(D // LANES):
                    sl = pl.ds(jj * LANES, LANES)
                    rows[r][e, sl] = rows[r][e, sl] * wv

            pltpu.async_copy(rows[r], acc.at[metas[m].at[1]], sem_s[r],
                             add=True)

        def wait_scatter(r):
            pltpu.make_async_copy(rows[r], acc.at[metas[0].at[1]],
                                  sem_s[r]).wait()

        for m in range(NM):
            stage(m, m)
        for r in range(NB):
            gather(r, r, r)

        @pl.loop(0, n_chunks, step=NM)
        def _(i):
            for t in range(NM):
                j = i + t

                @pl.when(j < n_chunks)
                def _(j=j, t=t):
                    process(j, t, t % NB)

                if t >= 1:
                    jd = i + t - 1
                    jr = jd + NB
                    js = jr + NB

                    @pl.when(jd < n_chunks)
                    def _(jd=jd, t=t):
                        wait_scatter((t - 1) % NB)

                    @pl.when(jr < n_chunks)
                    def _(jr=jr, t=t):
                        gather(jr, (t - 1 + NB) % NM, (t - 1) % NB)

                    @pl.when(js < n_chunks)
                    def _(js=js, t=t):
                        stage(js, (t - 1) % NM)

            jd = i + NM - 1
            jr = jd + NB
            js = jr + NB

            @pl.when(jd < n_chunks)
            def _():
                wait_scatter((NM - 1) % NB)

            @pl.when(jr < n_chunks)
            def _():
                gather(jr, (NM - 1 + NB) % NM, (NM - 1) % NB)

            @pl.when(js < n_chunks)
            def _():
                stage(js, (NM - 1) % NM)

        plsc.subcore_barrier()
        pltpu.sync_copy(acc.at[pl.ds(row0, rpt)],
                        out_hbm.at[c, pl.ds(row0, rpt)])

    return edge_kernel


def _mm_body(x_ref, w_ref, o_ref):
    o_ref[...] = jnp.dot(x_ref[...], w_ref[...],
                         preferred_element_type=jnp.float32,
                         precision=lax.Precision.HIGHEST)


def _mm(x, W):
    N, K = x.shape
    M = W.shape[1]
    return pl.pallas_call(
        _mm_body,
        grid=(N // ROW_BLK,),
        in_specs=[
            pl.BlockSpec((ROW_BLK, K), lambda i: (i, 0)),
            pl.BlockSpec((K, M), lambda i: (0, 0)),
        ],
        out_specs=pl.BlockSpec((ROW_BLK, M), lambda i: (i, 0)),
        out_shape=jax.ShapeDtypeStruct((N, M), jnp.float32),
    )(x, W)


def _mm1s_body(degp_ref, x_ref, w_ref, xw_ref, xs_ref, dis_ref):
    deg = degp_ref[0, :] + degp_ref[1, :] + 1.0
    dis = lax.rsqrt(deg)[:, None]
    dis_ref[...] = dis
    xw = jnp.dot(x_ref[...], w_ref[...],
                 preferred_element_type=jnp.float32,
                 precision=lax.Precision.HIGHEST)
    xw_ref[...] = xw
    xs_ref[...] = xw * dis


def _mm1s(degp, xp, W):
    NP, K = xp.shape
    M = W.shape[1]
    blk = 512
    return pl.pallas_call(
        _mm1s_body,
        grid=(NP // blk,),
        in_specs=[
            pl.BlockSpec((NC, blk), lambda i: (0, i)),
            pl.BlockSpec((blk, K), lambda i: (i, 0)),
            pl.BlockSpec((K, M), lambda i: (0, 0)),
        ],
        out_specs=[
            pl.BlockSpec((blk, M), lambda i: (i, 0)),
            pl.BlockSpec((blk, M), lambda i: (i, 0)),
            pl.BlockSpec((blk, 1), lambda i: (i, 0)),
        ],
        out_shape=[
            jax.ShapeDtypeStruct((NP, M), jnp.float32),
            jax.ShapeDtypeStruct((NP, M), jnp.float32),
            jax.ShapeDtypeStruct((NP, 1), jnp.float32),
        ],
    )(degp, xp, W)


def _comb2_body(ep_ref, xw1_ref, dis_ref, b1_ref, w2_ref, xw2_ref, xs2_ref):
    dis = dis_ref[...]
    h = (ep_ref[0] + ep_ref[1]) * dis + xw1_ref[...] * (dis * dis) + b1_ref[...]
    h = jnp.maximum(h, 0.0)
    xw2 = jnp.dot(h, w2_ref[...], preferred_element_type=jnp.float32,
                  precision=lax.Precision.HIGHEST)
    xw2_ref[...] = xw2
    pad = jnp.zeros_like(xw2)
    xs2_ref[...] = jnp.concatenate([xw2 * dis, pad], axis=1)


def _comb2(ep, xw1, dis, b1, W2):
    NProws, D = xw1.shape
    M = W2.shape[1]
    blk = 512
    return pl.pallas_call(
        _comb2_body,
        grid=(NProws // blk,),
        in_specs=[
            pl.BlockSpec((NC, blk, ep.shape[2]), lambda i: (0, i, 0)),
            pl.BlockSpec((blk, D), lambda i: (i, 0)),
            pl.BlockSpec((blk, 1), lambda i: (i, 0)),
            pl.BlockSpec((1, D), lambda i: (0, 0)),
            pl.BlockSpec((D, M), lambda i: (0, 0)),
        ],
        out_specs=[
            pl.BlockSpec((blk, M), lambda i: (i, 0)),
            pl.BlockSpec((blk, 2 * M), lambda i: (i, 0)),
        ],
        out_shape=[
            jax.ShapeDtypeStruct((NProws, M), jnp.float32),
            jax.ShapeDtypeStruct((NProws, 2 * M), jnp.float32),
        ],
    )(ep, xw1, dis, b1, W2)


def _final_body(ep_ref, xw2_ref, dis_ref, b2_ref, o_ref):
    dis = dis_ref[...]
    m_out = xw2_ref.shape[1]
    esum = ep_ref[0, :, :m_out] + ep_ref[1, :, :m_out]
    o = esum * dis + xw2_ref[...] * (dis * dis) + b2_ref[...]
    m = jnp.max(o, axis=1, keepdims=True)
    z = o - m
    o_ref[...] = z - jnp.log(jnp.sum(jnp.exp(z), axis=1, keepdims=True))


def _final(ep, xw2, dis, b2, N):
    M = xw2.shape[1]
    return pl.pallas_call(
        _final_body,
        grid=(N // ROW_BLK,),
        in_specs=[
            pl.BlockSpec((NC, ROW_BLK, ep.shape[2]), lambda i: (0, i, 0)),
            pl.BlockSpec((ROW_BLK, M), lambda i: (i, 0)),
            pl.BlockSpec((ROW_BLK, 1), lambda i: (i, 0)),
            pl.BlockSpec((1, M), lambda i: (0, 0)),
        ],
        out_specs=pl.BlockSpec((ROW_BLK, M), lambda i: (i, 0)),
        out_shape=jax.ShapeDtypeStruct((N, M), jnp.float32),
    )(ep, xw2, dis, b2)


def kernel(x, edge_index, edge_weight, W1, b1, W2, b2):
    N = x.shape[0]
    E = edge_index.shape[1]
    src = edge_index[0].astype(jnp.int32)
    dst = edge_index[1].astype(jnp.int32)
    ew = edge_weight.astype(jnp.float32)

    C = EDGE_CHUNK
    n_chunks = E // (NC * NS) // C
    ew_bits = lax.bitcast_convert_type(ew, jnp.int32)
    meta3 = jnp.stack(
        [src.reshape(NC * NS, n_chunks, C),
         dst.reshape(NC * NS, n_chunks, C),
         ew_bits.reshape(NC * NS, n_chunks, C)], axis=2)

    NP = _pad_rows(N)
    xp = jnp.pad(x, ((0, NP - N), (0, 0)))

    degp = _deg_kernel(N, E)(dst, ew)
    xw1, xs1, dis = _mm1s(degp, xp, W1)
    ep1 = _edge_kernel(N, E, W1.shape[1])(meta3, xs1)
    xw2, xs2 = _comb2(ep1, xw1, dis, b1.reshape(1, -1), W2)
    ep2 = _edge_kernel(N, E, xs2.shape[1])(meta3, xs2)
    return _final(ep2, xw2, dis, b2.reshape(1, -1), N)
